# Initial kernel scaffold; baseline (speedup 1.0000x reference)
import jax
import jax.numpy as jnp
from jax.experimental import pallas as pl


def kernel(x, edge_index, W1, b1, W2, b2):
    raise NotImplementedError("write your pallas kernel here")


# speedup vs baseline: 9.0367x; 9.0367x over previous
import functools

import jax
import jax.numpy as jnp
from jax import lax
from jax.experimental import pallas as pl
from jax.experimental.pallas import tpu as pltpu
from jax.experimental.pallas import tpu_sc as plsc

N = 10000
E = 320000
D = 128

NC = 2
NS = 16
B = 128
CH = 80
EPAD = NC * NS * CH * B
NPAD = 10240
RPT = NPAD // NS
BM = 1024

_mesh = plsc.VectorSubcoreMesh(core_axis_name="c", subcore_axis_name="s")
_sc_params = pltpu.CompilerParams(needs_layout_passes=False)


@functools.partial(
    pl.kernel,
    mesh=_mesh,
    out_type=jax.ShapeDtypeStruct((NC * NS, NPAD), jnp.float32),
    scratch_types=[
        pltpu.VMEM((CH, B), jnp.int32),
        pltpu.VMEM((NPAD,), jnp.float32),
    ],
    compiler_params=_sc_params,
)
def _deg_call(dst_hbm, out_hbm, idx_v, cnt_v):
    cid = lax.axis_index("c")
    sid = lax.axis_index("s")
    zero16 = jnp.zeros((16,), jnp.float32)

    def zbody(i, carry):
        cnt_v[pl.ds(i * 16, 16)] = zero16
        return carry

    lax.fori_loop(0, NPAD // 16, zbody, 0)

    pltpu.sync_copy(dst_hbm.at[cid, sid], idx_v)
    ones = jnp.ones((16,), jnp.float32)

    def body(k, carry):
        for j in range(B // 16):
            idx = idx_v[k, pl.ds(j * 16, 16)]
            plsc.addupdate_scatter(cnt_v, [idx], ones)
        return carry

    lax.fori_loop(0, CH, body, 0)
    pltpu.sync_copy(cnt_v, out_hbm.at[cid * NS + sid])


@functools.partial(
    pl.kernel,
    mesh=_mesh,
    out_type=jax.ShapeDtypeStruct((NC, NPAD, D), jnp.float32),
    scratch_types=[
        pltpu.VMEM((2, B), jnp.int32),
        pltpu.VMEM((2, B), jnp.int32),
        pltpu.VMEM((B, D), jnp.float32),
        pltpu.VMEM((B, D), jnp.float32),
        pltpu.VMEM_SHARED((NPAD, D), jnp.float32),
        pltpu.SemaphoreType.DMA,
        pltpu.SemaphoreType.DMA,
        pltpu.SemaphoreType.DMA,
        pltpu.SemaphoreType.DMA,
    ],
    compiler_params=_sc_params,
)
def _edge_call(g_hbm, sd_hbm, out_hbm,
               sdi0, sdi1, rows0, rows1, acc, semi0, semi1, semg0, semg1):
    cid = lax.axis_index("c")
    sid = lax.axis_index("s")
    zero16 = jnp.zeros((16,), jnp.float32)

    def zbody(i, carry):
        for j in range(D // 16):
            rows0[i, pl.ds(j * 16, 16)] = zero16
        return carry

    lax.fori_loop(0, B, zbody, 0)
    for t in range(RPT // B):
        pltpu.sync_copy(rows0, acc.at[pl.ds(sid * RPT + t * B, B)])
    plsc.subcore_barrier()

    pltpu.sync_copy(sd_hbm.at[cid, sid, 0], sdi0)
    pltpu.async_copy(sd_hbm.at[cid, sid, 1], sdi1, semi1)
    pltpu.async_copy(g_hbm.at[sdi0.at[0]], rows0, semg0)

    def half(k, sdiA, sdiB, rowsA, rowsB, semiA, semiB, semgA, semgB):
        pltpu.make_async_copy(
            sd_hbm.at[cid, sid, jnp.minimum(k + 1, CH - 1)], sdiB, semiB
        ).wait()
        pltpu.async_copy(g_hbm.at[sdiB.at[0]], rowsB, semgB)
        pltpu.make_async_copy(g_hbm.at[sdiA.at[0]], rowsA, semgA).wait()
        pltpu.sync_copy(rowsA, acc.at[sdiA.at[1]], add=True)
        pltpu.async_copy(
            sd_hbm.at[cid, sid, jnp.minimum(k + 2, CH - 1)], sdiA, semiA
        )

    def body(k2, carry):
        k = 2 * k2
        half(k, sdi0, sdi1, rows0, rows1, semi0, semi1, semg0, semg1)
        half(k + 1, sdi1, sdi0, rows1, rows0, semi1, semi0, semg1, semg0)
        return carry

    lax.fori_loop(0, CH // 2, body, 0)
    pltpu.make_async_copy(sd_hbm.at[cid, sid, CH - 1], sdi1, semi1).wait()
    pltpu.make_async_copy(g_hbm.at[sdi0.at[0]], rows0, semg0).wait()
    plsc.subcore_barrier()
    pltpu.sync_copy(acc.at[pl.ds(sid * RPT, RPT)],
                    out_hbm.at[cid, pl.ds(sid * RPT, RPT)])


def _mm1_body(x_ref, w_ref, dp_ref, g_ref, dinv_ref):
    dinv = lax.rsqrt(jnp.sum(dp_ref[...], axis=0) + 1.0)
    g_ref[...] = jnp.dot(x_ref[...], w_ref[...],
                         preferred_element_type=jnp.float32) * dinv
    dinv_ref[...] = dinv


def _mm2_body(s_ref, g_ref, dinv_ref, b_ref, w_ref, out_ref):
    s = s_ref[...]
    dinv = dinv_ref[...]
    h = jnp.maximum(dinv * (s[0] + s[1] + g_ref[...]) + b_ref[...], 0.0)
    out_ref[...] = jnp.dot(h, w_ref[...],
                           preferred_element_type=jnp.float32) * dinv


def _out_body(s_ref, g_ref, dinv_ref, b_ref, out_ref):
    s = s_ref[...]
    out_ref[...] = dinv_ref[...] * (s[0] + s[1] + g_ref[...]) + b_ref[...]


def _tc_mm1(x_pad, W1, deg_parts):
    return pl.pallas_call(
        _mm1_body,
        grid=(NPAD // BM,),
        in_specs=[
            pl.BlockSpec((BM, D), lambda i: (i, 0)),
            pl.BlockSpec((D, D), lambda i: (0, 0)),
            pl.BlockSpec((NC * NS, BM, 1), lambda i: (0, i, 0)),
        ],
        out_specs=[
            pl.BlockSpec((BM, D), lambda i: (i, 0)),
            pl.BlockSpec((BM, 1), lambda i: (i, 0)),
        ],
        out_shape=[
            jax.ShapeDtypeStruct((NPAD, D), jnp.float32),
            jax.ShapeDtypeStruct((NPAD, 1), jnp.float32),
        ],
    )(x_pad, W1, deg_parts)


def _tc_mm2(s1, g1, dinv, b1, W2):
    return pl.pallas_call(
        _mm2_body,
        grid=(NPAD // BM,),
        in_specs=[
            pl.BlockSpec((NC, BM, D), lambda i: (0, i, 0)),
            pl.BlockSpec((BM, D), lambda i: (i, 0)),
            pl.BlockSpec((BM, 1), lambda i: (i, 0)),
            pl.BlockSpec((1, D), lambda i: (0, 0)),
            pl.BlockSpec((D, D), lambda i: (0, 0)),
        ],
        out_specs=pl.BlockSpec((BM, D), lambda i: (i, 0)),
        out_shape=jax.ShapeDtypeStruct((NPAD, D), jnp.float32),
    )(s1, g1, dinv, b1, W2)


def _tc_out(s2, g2, dinv, b2):
    return pl.pallas_call(
        _out_body,
        grid=(NPAD // BM,),
        in_specs=[
            pl.BlockSpec((NC, BM, D), lambda i: (0, i, 0)),
            pl.BlockSpec((BM, D), lambda i: (i, 0)),
            pl.BlockSpec((BM, 1), lambda i: (i, 0)),
            pl.BlockSpec((1, D), lambda i: (0, 0)),
        ],
        out_specs=pl.BlockSpec((BM, D), lambda i: (i, 0)),
        out_shape=jax.ShapeDtypeStruct((NPAD, D), jnp.float32),
    )(s2, g2, dinv, b2)


def kernel(x, edge_index, W1, b1, W2, b2):
    pad = jnp.full((EPAD - E,), N, jnp.int32)
    src_p = jnp.concatenate([edge_index[0], pad]).reshape(NC, NS, CH, B)
    dst_p = jnp.concatenate([edge_index[1], pad]).reshape(NC, NS, CH, B)
    sd_p = jnp.stack([src_p, dst_p], axis=3)
    x_pad = jnp.pad(x, ((0, NPAD - N), (0, 0)))

    deg_parts = _deg_call(dst_p).reshape(NC * NS, NPAD, 1)
    g1, dinv = _tc_mm1(x_pad, W1, deg_parts)
    s1 = _edge_call(g1, sd_p)
    g2 = _tc_mm2(s1, g1, dinv, b1.reshape(1, D), W2)
    s2 = _edge_call(g2, sd_p)
    out = _tc_out(s2, g2, dinv, b2.reshape(1, D))
    return out[:N]


# speedup vs baseline: 31.6788x; 3.5056x over previous
import functools

import jax
import jax.numpy as jnp
from jax import lax
from jax.experimental import pallas as pl
from jax.experimental.pallas import tpu as pltpu
from jax.experimental.pallas import tpu_sc as plsc

N = 10000
E = 320000
D = 128

NC = 2
NS = 16
B = 128
NCHUNK = E // B
PAIRS = NCHUNK // 2
P0 = PAIRS // 2
P1 = PAIRS - P0
NP = 10240
RPT = NP // NS
NROW = 640
BM = 1000

_mesh = plsc.VectorSubcoreMesh(core_axis_name="c", subcore_axis_name="s")
_sc_params = pltpu.CompilerParams(needs_layout_passes=False)


@functools.partial(
    pl.kernel,
    mesh=_mesh,
    out_type=jax.ShapeDtypeStruct((NC * NS, NP), jnp.float32),
    scratch_types=[
        pltpu.VMEM((E // (NC * NS),), jnp.int32),
        pltpu.VMEM((NP,), jnp.float32),
    ],
    compiler_params=_sc_params,
)
def _deg_call(edge_hbm, out_hbm, idx_v, cnt_v):
    cid = lax.axis_index("c")
    sid = lax.axis_index("s")
    w = cid * NS + sid
    zero16 = jnp.zeros((16,), jnp.float32)

    def zbody(i, carry):
        cnt_v[pl.ds(i * 16, 16)] = zero16
        return carry

    lax.fori_loop(0, NP // 16, zbody, 0)
    ew = E // (NC * NS)
    pltpu.sync_copy(edge_hbm.at[pl.ds(E + w * ew, ew)], idx_v)
    ones = jnp.ones((16,), jnp.float32)

    def body(k, carry):
        idx = idx_v[pl.ds(k * 16, 16)]
        plsc.addupdate_scatter(cnt_v, [idx], ones)
        return carry

    lax.fori_loop(0, ew // 16, body, 0)
    pltpu.sync_copy(cnt_v, out_hbm.at[w])


@functools.partial(
    pl.kernel,
    mesh=_mesh,
    out_type=jax.ShapeDtypeStruct((NC, NP, D), jnp.float32),
    scratch_types=[
        pltpu.VMEM((B,), jnp.int32),
        pltpu.VMEM((B,), jnp.int32),
        pltpu.VMEM((B,), jnp.int32),
        pltpu.VMEM((B,), jnp.int32),
        pltpu.VMEM((B, D), jnp.float32),
        pltpu.VMEM((B, D), jnp.float32),
        pltpu.VMEM_SHARED((NP, D), jnp.float32),
        pltpu.SemaphoreType.DMA,
        pltpu.SemaphoreType.DMA,
        pltpu.SemaphoreType.DMA,
        pltpu.SemaphoreType.DMA,
        pltpu.SemaphoreType.DMA,
        pltpu.SemaphoreType.DMA,
    ],
    compiler_params=_sc_params,
)
def _edge_call(g_hbm, edge_hbm, out_hbm,
               si0, si1, di0, di1, rows0, rows1, acc,
               ss0, ss1, sd0, sd1, sg0, sg1):
    cid = lax.axis_index("c")
    sid = lax.axis_index("s")
    zero16 = jnp.zeros((16,), jnp.float32)

    def zbody(i, carry):
        for j in range(D // 16):
            rows0[i, pl.ds(j * 16, 16)] = zero16
        return carry

    lax.fori_loop(0, B, zbody, 0)
    for t in range(RPT // B):
        pltpu.sync_copy(rows0, acc.at[pl.ds(sid * RPT + t * B, B)])
    plsc.subcore_barrier()

    pc = jnp.where(cid == 0, P0, P1)
    pbase = jnp.where(cid == 0, 0, P0)
    lo = pbase + (sid * pc) // NS
    hi = pbase + ((sid + 1) * pc) // NS
    npair = hi - lo
    k0 = 2 * lo
    klast = 2 * hi - 1

    def fetch_idx(k, si, di, ssem, dsem):
        pltpu.async_copy(edge_hbm.at[pl.ds(k * B, B)], si, ssem)
        pltpu.async_copy(edge_hbm.at[pl.ds(E + k * B, B)], di, dsem)

    def wait_idx(k, si, di, ssem, dsem):
        pltpu.make_async_copy(edge_hbm.at[pl.ds(k * B, B)], si, ssem).wait()
        pltpu.make_async_copy(edge_hbm.at[pl.ds(E + k * B, B)], di, dsem).wait()

    fetch_idx(k0, si0, di0, ss0, sd0)
    wait_idx(k0, si0, di0, ss0, sd0)
    pltpu.async_copy(g_hbm.at[si0], rows0, sg0)
    fetch_idx(k0 + 1, si1, di1, ss1, sd1)

    def half(k, siA, diA, rowsA, ssA, sdA, sgA, siB, diB, rowsB, ssB, sdB, sgB):
        wait_idx(jnp.minimum(k + 1, klast), siB, diB, ssB, sdB)
        pltpu.async_copy(g_hbm.at[siB], rowsB, sgB)
        pltpu.make_async_copy(g_hbm.at[siA], rowsA, sgA).wait()
        pltpu.sync_copy(rowsA, acc.at[diA], add=True)
        fetch_idx(jnp.minimum(k + 2, klast), siA, diA, ssA, sdA)

    def body(j, carry):
        k = k0 + 2 * j
        half(k, si0, di0, rows0, ss0, sd0, sg0, si1, di1, rows1, ss1, sd1, sg1)
        half(k + 1, si1, di1, rows1, ss1, sd1, sg1,
             si0, di0, rows0, ss0, sd0, sg0)
        return carry

    lax.fori_loop(0, npair, body, 0)
    pltpu.make_async_copy(g_hbm.at[si0], rows0, sg0).wait()
    wait_idx(klast, si1, di1, ss1, sd1)
    plsc.subcore_barrier()
    pltpu.sync_copy(acc.at[pl.ds(sid * RPT, RPT)],
                    out_hbm.at[cid, pl.ds(sid * RPT, RPT)])


def _dinv_body(dp_ref, out_ref):
    out_ref[...] = lax.rsqrt(jnp.sum(dp_ref[...], axis=0, keepdims=True) + 1.0)


def _tc_dinv(parts):
    return pl.pallas_call(
        _dinv_body,
        grid=(1,),
        in_specs=[pl.BlockSpec((NC * NS, NP), lambda i: (0, 0))],
        out_specs=pl.BlockSpec((1, NP), lambda i: (0, 0)),
        out_shape=jax.ShapeDtypeStruct((1, NP), jnp.float32),
    )(parts)


def _mm1_body(x_ref, w_ref, dinv_ref, g_ref):
    g_ref[...] = jnp.dot(x_ref[...], w_ref[...],
                         precision=lax.Precision.HIGHEST,
                         preferred_element_type=jnp.float32) * dinv_ref[...]


def _mm2_body(s_ref, g_ref, dinv_ref, b_ref, w_ref, out_ref):
    s = s_ref[...]
    dinv = dinv_ref[...]
    h = jnp.maximum(dinv * (s[0] + s[1] + g_ref[...]) + b_ref[...], 0.0)
    out_ref[...] = jnp.dot(h, w_ref[...],
                           precision=lax.Precision.HIGHEST,
                           preferred_element_type=jnp.float32) * dinv


def _out_body(s_ref, g_ref, dinv_ref, b_ref, out_ref):
    s = s_ref[...]
    out_ref[...] = dinv_ref[...] * (s[0] + s[1] + g_ref[...]) + b_ref[...]


def _tc_mm1(x, W1, dinv):
    return pl.pallas_call(
        _mm1_body,
        grid=(N // BM,),
        in_specs=[
            pl.BlockSpec((BM, D), lambda i: (i, 0)),
            pl.BlockSpec((D, D), lambda i: (0, 0)),
            pl.BlockSpec((BM, 1), lambda i: (i, 0)),
        ],
        out_specs=pl.BlockSpec((BM, D), lambda i: (i, 0)),
        out_shape=jax.ShapeDtypeStruct((N, D), jnp.float32),
    )(x, W1, dinv)


def _tc_mm2(s1, g1, dinv, b1, W2):
    return pl.pallas_call(
        _mm2_body,
        grid=(N // BM,),
        in_specs=[
            pl.BlockSpec((NC, BM, D), lambda i: (0, i, 0)),
            pl.BlockSpec((BM, D), lambda i: (i, 0)),
            pl.BlockSpec((BM, 1), lambda i: (i, 0)),
            pl.BlockSpec((1, D), lambda i: (0, 0)),
            pl.BlockSpec((D, D), lambda i: (0, 0)),
        ],
        out_specs=pl.BlockSpec((BM, D), lambda i: (i, 0)),
        out_shape=jax.ShapeDtypeStruct((N, D), jnp.float32),
    )(s1, g1, dinv, b1, W2)


def _tc_out(s2, g2, dinv, b2):
    return pl.pallas_call(
        _out_body,
        grid=(N // BM,),
        in_specs=[
            pl.BlockSpec((NC, BM, D), lambda i: (0, i, 0)),
            pl.BlockSpec((BM, D), lambda i: (i, 0)),
            pl.BlockSpec((BM, 1), lambda i: (i, 0)),
            pl.BlockSpec((1, D), lambda i: (0, 0)),
        ],
        out_specs=pl.BlockSpec((BM, D), lambda i: (i, 0)),
        out_shape=jax.ShapeDtypeStruct((N, D), jnp.float32),
    )(s2, g2, dinv, b2)


def kernel(x, edge_index, W1, b1, W2, b2):
    edge_flat = edge_index.reshape(2 * E)
    parts = _deg_call(edge_flat)
    dinv = _tc_dinv(parts).reshape(NP, 1)[:N]
    g1 = _tc_mm1(x, W1, dinv)
    s1 = _edge_call(g1, edge_flat)
    g2 = _tc_mm2(s1, g1, dinv, b1.reshape(1, D), W2)
    s2 = _edge_call(g2, edge_flat)
    out = _tc_out(s2, g2, dinv, b2.reshape(1, D))
    return out


# speedup vs baseline: 35.9811x; 1.1358x over previous
import functools

import jax
import jax.numpy as jnp
from jax import lax
from jax.experimental import pallas as pl
from jax.experimental.pallas import tpu as pltpu
from jax.experimental.pallas import tpu_sc as plsc

N = 10000
E = 320000
D = 128

NC = 2
NS = 16
B = 128
NCHUNK = E // B
PAIRS = NCHUNK // 2
P0 = PAIRS // 2
P1 = PAIRS - P0
NP = 10240
RPT = NP // NS
BM = 2000

_mesh = plsc.VectorSubcoreMesh(core_axis_name="c", subcore_axis_name="s")
_sc_params = pltpu.CompilerParams(needs_layout_passes=False)


@functools.partial(
    pl.kernel,
    mesh=_mesh,
    out_type=jax.ShapeDtypeStruct((NC * NS, NP), jnp.float32),
    scratch_types=[
        pltpu.VMEM((E // (NC * NS),), jnp.int32),
        pltpu.VMEM((NP,), jnp.float32),
    ],
    compiler_params=_sc_params,
)
def _deg_call(edge_hbm, out_hbm, idx_v, cnt_v):
    cid = lax.axis_index("c")
    sid = lax.axis_index("s")
    w = cid * NS + sid
    zero16 = jnp.zeros((16,), jnp.float32)

    def zbody(i, carry):
        cnt_v[pl.ds(i * 16, 16)] = zero16
        return carry

    lax.fori_loop(0, NP // 16, zbody, 0)
    ew = E // (NC * NS)
    pltpu.sync_copy(edge_hbm.at[pl.ds(E + w * ew, ew)], idx_v)
    ones = jnp.ones((16,), jnp.float32)

    def body(k, carry):
        idx = idx_v[pl.ds(k * 16, 16)]
        plsc.addupdate_scatter(cnt_v, [idx], ones)
        return carry

    lax.fori_loop(0, ew // 16, body, 0)
    pltpu.sync_copy(cnt_v, out_hbm.at[w])


@functools.partial(
    pl.kernel,
    mesh=_mesh,
    out_type=jax.ShapeDtypeStruct((NC, NP, D), jnp.float32),
    scratch_types=[
        pltpu.VMEM((B,), jnp.int32),
        pltpu.VMEM((B,), jnp.int32),
        pltpu.VMEM((B,), jnp.int32),
        pltpu.VMEM((B,), jnp.int32),
        pltpu.VMEM((B,), jnp.int32),
        pltpu.VMEM((B,), jnp.int32),
        pltpu.VMEM((B, D), jnp.float32),
        pltpu.VMEM((B, D), jnp.float32),
        pltpu.VMEM_SHARED((NP, D), jnp.float32),
        pltpu.SemaphoreType.DMA,
        pltpu.SemaphoreType.DMA,
        pltpu.SemaphoreType.DMA,
        pltpu.SemaphoreType.DMA,
        pltpu.SemaphoreType.DMA,
        pltpu.SemaphoreType.DMA,
        pltpu.SemaphoreType.DMA,
        pltpu.SemaphoreType.DMA,
        pltpu.SemaphoreType.DMA,
        pltpu.SemaphoreType.DMA,
    ],
    compiler_params=_sc_params,
)
def _edge_call(g_hbm, edge_hbm, out_hbm,
               si0, si1, di0, di1, di2, di3, rows0, rows1, acc,
               ss0, ss1, sd0, sd1, sd2, sd3, sg0, sg1, sc0, sc1):
    cid = lax.axis_index("c")
    sid = lax.axis_index("s")
    zero16 = jnp.zeros((16,), jnp.float32)

    QUADS = NCHUNK // 4
    Q0 = QUADS // 2
    qc = jnp.where(cid == 0, Q0, QUADS - Q0)
    qbase = jnp.where(cid == 0, 0, Q0)
    qlo = qbase + (sid * qc) // NS
    qhi = qbase + ((sid + 1) * qc) // NS
    nquad = qhi - qlo
    k0 = 4 * qlo
    klast = 4 * qhi - 1

    def fetch_src(k, si, ssem):
        pltpu.async_copy(edge_hbm.at[pl.ds(k * B, B)], si, ssem)

    def wait_src(k, si, ssem):
        pltpu.make_async_copy(edge_hbm.at[pl.ds(k * B, B)], si, ssem).wait()

    def fetch_dst(k, di, dsem):
        pltpu.async_copy(edge_hbm.at[pl.ds(E + k * B, B)], di, dsem)

    def wait_dst(k, di, dsem):
        pltpu.make_async_copy(edge_hbm.at[pl.ds(E + k * B, B)], di,
                              dsem).wait()

    fetch_src(k0, si0, ss0)
    fetch_src(k0 + 1, si1, ss1)
    fetch_dst(k0, di0, sd0)
    fetch_dst(k0 + 1, di1, sd1)
    fetch_dst(k0 + 2, di2, sd2)

    def zbody(i, carry):
        for j in range(D // 16):
            rows1[i, pl.ds(j * 16, 16)] = zero16
        return carry

    lax.fori_loop(0, B, zbody, 0)
    wait_src(k0, si0, ss0)
    pltpu.async_copy(g_hbm.at[si0], rows0, sg0)
    for t in range(RPT // B):
        pltpu.async_copy(rows1, acc.at[pl.ds(sid * RPT + t * B, B)], sc0)
    for t in range(RPT // B):
        pltpu.make_async_copy(rows1, acc.at[pl.ds(sid * RPT + t * B, B)],
                              sc0).wait()
    plsc.subcore_barrier()

    SIB = (si0, si1)
    SSB = (ss0, ss1)
    DIB = (di0, di1, di2, di3)
    SDB = (sd0, sd1, sd2, sd3)
    RWB = (rows0, rows1)
    SGB = (sg0, sg1)
    SCB = (sc0, sc1)

    def half(k, m, wait_prev):
        wait_src(jnp.minimum(k + 1, klast), SIB[(m + 1) % 2], SSB[(m + 1) % 2])
        wait_dst(jnp.minimum(k + 1, klast), DIB[(m + 1) % 4], SDB[(m + 1) % 4])

        @pl.when(wait_prev)
        def _():
            pltpu.make_async_copy(RWB[(m + 1) % 2],
                                  acc.at[DIB[(m + 3) % 4]],
                                  SCB[(m + 1) % 2]).wait()

        pltpu.async_copy(g_hbm.at[SIB[(m + 1) % 2]], RWB[(m + 1) % 2],
                         SGB[(m + 1) % 2])
        pltpu.make_async_copy(g_hbm.at[SIB[m % 2]], RWB[m % 2],
                              SGB[m % 2]).wait()
        pltpu.async_copy(RWB[m % 2], acc.at[DIB[m % 4]], SCB[m % 2],
                         add=True)
        fetch_src(jnp.minimum(k + 2, klast), SIB[m % 2], SSB[m % 2])
        fetch_dst(jnp.minimum(k + 3, klast), DIB[(m + 3) % 4],
                  SDB[(m + 3) % 4])

    def body(j, carry):
        k = k0 + 4 * j
        half(k, 0, j > 0)
        half(k + 1, 1, j >= 0)
        half(k + 2, 2, j >= 0)
        half(k + 3, 3, j >= 0)
        return carry

    lax.fori_loop(0, nquad, body, 0)
    pltpu.make_async_copy(rows1, acc.at[di3], sc1).wait()
    pltpu.make_async_copy(g_hbm.at[si0], rows0, sg0).wait()
    wait_src(klast, si1, ss1)
    wait_dst(klast, di1, sd1)
    wait_dst(klast, di2, sd2)
    plsc.subcore_barrier()
    pltpu.sync_copy(acc.at[pl.ds(sid * RPT, RPT)],
                    out_hbm.at[cid, pl.ds(sid * RPT, RPT)])


def _dinv_body(dp_ref, out_ref):
    s = lax.rsqrt(jnp.sum(dp_ref[...], axis=0) + 1.0)
    out_ref[...] = s.reshape(NP, 1)


def _tc_dinv(parts):
    return pl.pallas_call(
        _dinv_body,
        grid=(1,),
        in_specs=[pl.BlockSpec((NC * NS, NP), lambda i: (0, 0))],
        out_specs=pl.BlockSpec((NP, 1), lambda i: (0, 0)),
        out_shape=jax.ShapeDtypeStruct((NP, 1), jnp.float32),
    )(parts)


def _mm1_body(x_ref, w_ref, dinv_ref, g_ref):
    g_ref[...] = jnp.dot(x_ref[...], w_ref[...],
                         precision=lax.Precision.HIGHEST,
                         preferred_element_type=jnp.float32) * dinv_ref[...]


def _mm2_body(s_ref, g_ref, dinv_ref, b_ref, w_ref, out_ref):
    s = s_ref[...]
    dinv = dinv_ref[...]
    h = jnp.maximum(dinv * (s[0] + s[1] + g_ref[...]) + b_ref[...], 0.0)
    out_ref[...] = jnp.dot(h, w_ref[...],
                           precision=lax.Precision.HIGHEST,
                           preferred_element_type=jnp.float32) * dinv


def _out_body(s_ref, g_ref, dinv_ref, b_ref, out_ref):
    s = s_ref[...]
    out_ref[...] = dinv_ref[...] * (s[0] + s[1] + g_ref[...]) + b_ref[...]


def _tc_mm1(x, W1, dinv):
    return pl.pallas_call(
        _mm1_body,
        grid=(N // BM,),
        in_specs=[
            pl.BlockSpec((BM, D), lambda i: (i, 0)),
            pl.BlockSpec((D, D), lambda i: (0, 0)),
            pl.BlockSpec((BM, 1), lambda i: (i, 0)),
        ],
        out_specs=pl.BlockSpec((BM, D), lambda i: (i, 0)),
        out_shape=jax.ShapeDtypeStruct((N, D), jnp.float32),
    )(x, W1, dinv)


def _tc_mm2(s1, g1, dinv, b1, W2):
    return pl.pallas_call(
        _mm2_body,
        grid=(N // BM,),
        in_specs=[
            pl.BlockSpec((NC, BM, D), lambda i: (0, i, 0)),
            pl.BlockSpec((BM, D), lambda i: (i, 0)),
            pl.BlockSpec((BM, 1), lambda i: (i, 0)),
            pl.BlockSpec((1, D), lambda i: (0, 0)),
            pl.BlockSpec((D, D), lambda i: (0, 0)),
        ],
        out_specs=pl.BlockSpec((BM, D), lambda i: (i, 0)),
        out_shape=jax.ShapeDtypeStruct((N, D), jnp.float32),
    )(s1, g1, dinv, b1, W2)


def _tc_out(s2, g2, dinv, b2):
    return pl.pallas_call(
        _out_body,
        grid=(N // BM,),
        in_specs=[
            pl.BlockSpec((NC, BM, D), lambda i: (0, i, 0)),
            pl.BlockSpec((BM, D), lambda i: (i, 0)),
            pl.BlockSpec((BM, 1), lambda i: (i, 0)),
            pl.BlockSpec((1, D), lambda i: (0, 0)),
        ],
        out_specs=pl.BlockSpec((BM, D), lambda i: (i, 0)),
        out_shape=jax.ShapeDtypeStruct((N, D), jnp.float32),
    )(s2, g2, dinv, b2)


def kernel(x, edge_index, W1, b1, W2, b2):
    edge_flat = edge_index.reshape(2 * E)
    parts = _deg_call(edge_flat)
    dinv = _tc_dinv(parts)[:N]
    g1 = _tc_mm1(x, W1, dinv)
    s1 = _edge_call(g1, edge_flat)
    g2 = _tc_mm2(s1, g1, dinv, b1.reshape(1, D), W2)
    s2 = _edge_call(g2, edge_flat)
    out = _tc_out(s2, g2, dinv, b2.reshape(1, D))
    return out
